# Initial kernel scaffold; baseline (speedup 1.0000x reference)
import jax
import jax.numpy as jnp
from jax.experimental import pallas as pl


def kernel(data, edge, W1, b1, W2, b2, W3, b3, Wl1, bl1, Wl2, bl2):
    raise NotImplementedError("write your pallas kernel here")


# speedup vs baseline: 33.4480x; 33.4480x over previous
import functools

import jax
import jax.numpy as jnp
from jax import lax
from jax.experimental import pallas as pl
from jax.experimental.pallas import tpu as pltpu
from jax.experimental.pallas import tpu_sc as plsc

N = 10000
E = 640000
NPAD = 10240
LANES = 16
NTILES = 16
EPT = E // NTILES
CHUNK = 2000
NCHUNKS = EPT // CHUNK
VECS = CHUNK // 16
SLICE = NPAD // NTILES

_MESH = plsc.VectorSubcoreMesh(
    core_axis_name="c", subcore_axis_name="s", num_cores=1)
_SC_PARAMS = pltpu.CompilerParams(
    needs_layout_passes=False, use_tc_tiling_on_sc=False)


def _make_count():
  @functools.partial(
      pl.kernel,
      out_type=jax.ShapeDtypeStruct((NPAD, LANES), jnp.float32),
      mesh=_MESH,
      compiler_params=_SC_PARAMS,
      scratch_types=[
          pltpu.VMEM((CHUNK,), jnp.int32),
          pltpu.VMEM((CHUNK, LANES), jnp.float32),
          pltpu.VMEM_SHARED((NPAD, LANES), jnp.float32),
      ],
  )
  def k(dst_hbm, zeros_hbm, out_hbm, dst_buf, vals, acc):
    wid = lax.axis_index("s")

    @pl.when(wid == 0)
    def _():
      pltpu.sync_copy(zeros_hbm, acc)

    pltpu.sync_copy(zeros_hbm.at[pl.ds(0, CHUNK)], vals)
    iota = lax.iota(jnp.int32, 16)
    zero16 = jnp.zeros((16,), jnp.int32)
    one16 = jnp.ones((16,), jnp.float32)

    @pl.loop(0, VECS)
    def _(i):
      plsc.store_scatter(vals, [i * 16 + iota, zero16], one16)

    plsc.subcore_barrier()
    ebase = wid * EPT

    @pl.loop(0, NCHUNKS)
    def _(j):
      pltpu.sync_copy(dst_hbm.at[pl.ds(ebase + j * CHUNK, CHUNK)], dst_buf)
      pltpu.sync_copy(vals, acc.at[dst_buf], add=True)

    plsc.subcore_barrier()
    nbase = wid * SLICE
    pltpu.sync_copy(acc.at[pl.ds(nbase, SLICE)],
                    out_hbm.at[pl.ds(nbase, SLICE)])

  return k


def _make_edge_apply(C):
  @functools.partial(
      pl.kernel,
      out_type=jax.ShapeDtypeStruct((NPAD, LANES), jnp.float32),
      mesh=_MESH,
      compiler_params=_SC_PARAMS,
      scratch_types=[
          pltpu.VMEM((NPAD, C), jnp.float32),
          pltpu.VMEM((CHUNK,), jnp.int32),
          pltpu.VMEM((CHUNK,), jnp.int32),
          pltpu.VMEM((CHUNK, LANES), jnp.float32),
          pltpu.VMEM_SHARED((NPAD, LANES), jnp.float32),
      ],
  )
  def k(src_hbm, dst_hbm, hw_hbm, zeros_hbm, out_hbm,
        hw_loc, src_buf, dst_buf, vals, acc):
    wid = lax.axis_index("s")

    @pl.when(wid == 0)
    def _():
      pltpu.sync_copy(zeros_hbm, acc)

    pltpu.sync_copy(zeros_hbm.at[pl.ds(0, CHUNK)], vals)
    pltpu.sync_copy(hw_hbm, hw_loc)
    plsc.subcore_barrier()
    iota = lax.iota(jnp.int32, 16)
    ebase = wid * EPT

    @pl.loop(0, NCHUNKS)
    def _(j):
      pltpu.sync_copy(src_hbm.at[pl.ds(ebase + j * CHUNK, CHUNK)], src_buf)
      pltpu.sync_copy(dst_hbm.at[pl.ds(ebase + j * CHUNK, CHUNK)], dst_buf)

      @pl.loop(0, VECS)
      def _(i):
        s = src_buf[pl.ds(i * 16, 16)]
        rows = i * 16 + iota
        for c in range(C):
          cvec = jnp.full((16,), c, jnp.int32)
          hv = plsc.load_gather(hw_loc, [s, cvec])
          plsc.store_scatter(vals, [rows, cvec], hv)

      pltpu.sync_copy(vals, acc.at[dst_buf], add=True)

    plsc.subcore_barrier()
    nbase = wid * SLICE
    pltpu.sync_copy(acc.at[pl.ds(nbase, SLICE)],
                    out_hbm.at[pl.ds(nbase, SLICE)])

  return k


_count = _make_count()
_edge_apply = {c: _make_edge_apply(c) for c in (1, 2, 4)}


_VMEM_SPEC = pl.BlockSpec(memory_space=pltpu.VMEM)


def _node_prep_body(deg_ref, x_ref, w_ref, dinv_ref, hw_ref):
  dinv = lax.rsqrt(deg_ref[:, 0:1] + 1.0)
  dinv_ref[...] = dinv
  h = jnp.dot(x_ref[...], w_ref[...], preferred_element_type=jnp.float32)
  hw_ref[...] = h * dinv


def _node_prep(deg, x_nm, Wp):
  return pl.pallas_call(
      _node_prep_body,
      out_shape=(
          jax.ShapeDtypeStruct((NPAD, 1), jnp.float32),
          jax.ShapeDtypeStruct((NPAD, LANES), jnp.float32),
      ),
      in_specs=[_VMEM_SPEC] * 3,
      out_specs=(_VMEM_SPEC,) * 2,
  )(deg, x_nm, Wp)


def _mix_body(acc_ref, hw_ref, dinv_ref, b_ref, w_ref, out_ref):
  x = (acc_ref[...] + hw_ref[...]) * dinv_ref[...] + b_ref[...]
  x = jnp.maximum(x, 0.0)
  h = jnp.dot(x, w_ref[...], preferred_element_type=jnp.float32)
  out_ref[...] = h * dinv_ref[...]


def _mix(acc, hw, dinv, bp, Wp):
  return pl.pallas_call(
      _mix_body,
      out_shape=jax.ShapeDtypeStruct((NPAD, LANES), jnp.float32),
      in_specs=[_VMEM_SPEC] * 5,
      out_specs=_VMEM_SPEC,
  )(acc, hw, dinv, bp, Wp)


def _final_body(acc_ref, hw_ref, dinv_ref, b_ref, out_ref):
  out_ref[...] = ((acc_ref[:, 0:1] + hw_ref[:, 0:1]) * dinv_ref[...]
                  + b_ref[...])


def _final(acc, hw, dinv, b3):
  return pl.pallas_call(
      _final_body,
      out_shape=jax.ShapeDtypeStruct((NPAD, 1), jnp.float32),
      in_specs=[_VMEM_SPEC] * 4,
      out_specs=_VMEM_SPEC,
  )(acc, hw, dinv, b3.reshape(1, 1))


_CB = 512


def _gemv1_body(x_ref, w_ref, b_ref, o_ref):
  o_ref[...] = jnp.tanh(
      jnp.dot(x_ref[...], w_ref[...], preferred_element_type=jnp.float32)
      + b_ref[...])


def _gemv1(x, W, b):
  kdim, mdim = W.shape
  ncol = pl.cdiv(mdim, _CB)
  return pl.pallas_call(
      _gemv1_body,
      grid=(ncol,),
      in_specs=[
          pl.BlockSpec((1, kdim), lambda i: (0, 0)),
          pl.BlockSpec((kdim, _CB), lambda i: (0, i)),
          pl.BlockSpec((1, _CB), lambda i: (0, i)),
      ],
      out_specs=pl.BlockSpec((1, _CB), lambda i: (0, i)),
      out_shape=jax.ShapeDtypeStruct((1, mdim), jnp.float32),
  )(x, W, b)


def _gemv2_body(x_ref, w_ref, b_ref, o_ref):
  o_ref[...] = jnp.tanh(
      jnp.dot(x_ref[...], w_ref[...], preferred_element_type=jnp.float32)
      + b_ref[...])


def _gemv2(x, W, b):
  return pl.pallas_call(
      _gemv2_body,
      out_shape=jax.ShapeDtypeStruct((1, W.shape[1]), jnp.float32),
  )(x, W, b)


def _pad_w(W):
  return jnp.zeros((LANES, LANES), jnp.float32).at[:W.shape[0],
                                                   :W.shape[1]].set(W)


def _pad_b(b):
  return jnp.zeros((1, LANES), jnp.float32).at[0, :b.shape[0]].set(b)


def kernel(data, edge, W1, b1, W2, b2, W3, b3, Wl1, bl1, Wl2, bl2):
  src = edge[0]
  dst = edge[1]
  zeros = jnp.zeros((NPAD, LANES), jnp.float32)
  x_nm = jnp.zeros((NPAD, LANES), jnp.float32).at[:N, :2].set(data)

  deg = _count(dst, zeros)
  dinv, hw1 = _node_prep(deg, x_nm, _pad_w(W1))

  acc1 = _edge_apply[4](src, dst, hw1[:, :4], zeros)
  hw2 = _mix(acc1, hw1, dinv, _pad_b(b1), _pad_w(W2))

  acc2 = _edge_apply[2](src, dst, hw2[:, :2], zeros)
  hw3 = _mix(acc2, hw2, dinv, _pad_b(b2), _pad_w(W3))

  acc3 = _edge_apply[1](src, dst, hw3[:, :1], zeros)
  v = _final(acc3, hw3, dinv, b3)

  x_vec = v.reshape(1, NPAD)[:, :N]
  hmid = _gemv1(x_vec, Wl1, bl1.reshape(1, -1))
  return _gemv2(hmid, Wl2, bl2.reshape(1, -1))


# speedup vs baseline: 40.1040x; 1.1990x over previous
import functools

import jax
import jax.numpy as jnp
from jax import lax
from jax.experimental import pallas as pl
from jax.experimental.pallas import tpu as pltpu
from jax.experimental.pallas import tpu_sc as plsc

N = 10000
E = 640000
NPAD = 10240
LANES = 16
NCORES = 2
NTILES = 16
NWORK = NCORES * NTILES
EPT = E // NWORK
CHUNK = 2000
NCHUNKS = EPT // CHUNK
VECS = CHUNK // 16
SLICE = NPAD // NTILES

_MESH = plsc.VectorSubcoreMesh(
    core_axis_name="c", subcore_axis_name="s", num_cores=NCORES)
_SC_PARAMS = pltpu.CompilerParams(
    needs_layout_passes=False, use_tc_tiling_on_sc=False)


def _make_count():
  @functools.partial(
      pl.kernel,
      out_type=jax.ShapeDtypeStruct((NCORES, NPAD, LANES), jnp.float32),
      mesh=_MESH,
      compiler_params=_SC_PARAMS,
      scratch_types=[
          pltpu.VMEM((CHUNK,), jnp.int32),
          pltpu.VMEM((CHUNK, LANES), jnp.float32),
          pltpu.VMEM_SHARED((NPAD, LANES), jnp.float32),
      ],
  )
  def k(dst_hbm, zeros_hbm, out_hbm, dst_buf, vals, acc):
    cid = lax.axis_index("c")
    sid = lax.axis_index("s")

    @pl.when(sid == 0)
    def _():
      pltpu.sync_copy(zeros_hbm, acc)

    pltpu.sync_copy(zeros_hbm.at[pl.ds(0, CHUNK)], vals)
    iota = lax.iota(jnp.int32, 16)
    zero16 = jnp.zeros((16,), jnp.int32)
    one16 = jnp.ones((16,), jnp.float32)

    @pl.loop(0, VECS)
    def _(i):
      plsc.store_scatter(vals, [i * 16 + iota, zero16], one16)

    plsc.subcore_barrier()
    ebase = (cid * NTILES + sid) * EPT

    @pl.loop(0, NCHUNKS)
    def _(j):
      pltpu.sync_copy(dst_hbm.at[pl.ds(ebase + j * CHUNK, CHUNK)], dst_buf)
      pltpu.sync_copy(vals, acc.at[dst_buf], add=True)

    plsc.subcore_barrier()
    nbase = sid * SLICE
    pltpu.sync_copy(acc.at[pl.ds(nbase, SLICE)],
                    out_hbm.at[cid, pl.ds(nbase, SLICE)])

  return k


def _make_edge_apply(C):
  @functools.partial(
      pl.kernel,
      out_type=jax.ShapeDtypeStruct((NCORES, NPAD, LANES), jnp.float32),
      mesh=_MESH,
      compiler_params=_SC_PARAMS,
      scratch_types=[
          pltpu.VMEM((NPAD, C), jnp.float32),
          pltpu.VMEM((CHUNK,), jnp.int32),
          pltpu.VMEM((CHUNK,), jnp.int32),
          pltpu.VMEM((CHUNK, LANES), jnp.float32),
          pltpu.VMEM_SHARED((NPAD, LANES), jnp.float32),
      ],
  )
  def k(src_hbm, dst_hbm, hw_hbm, zeros_hbm, out_hbm,
        hw_loc, src_buf, dst_buf, vals, acc):
    cid = lax.axis_index("c")
    sid = lax.axis_index("s")

    @pl.when(sid == 0)
    def _():
      pltpu.sync_copy(zeros_hbm, acc)

    pltpu.sync_copy(zeros_hbm.at[pl.ds(0, CHUNK)], vals)
    pltpu.sync_copy(hw_hbm, hw_loc)
    plsc.subcore_barrier()
    iota = lax.iota(jnp.int32, 16)
    ebase = (cid * NTILES + sid) * EPT

    @pl.loop(0, NCHUNKS)
    def _(j):
      pltpu.sync_copy(src_hbm.at[pl.ds(ebase + j * CHUNK, CHUNK)], src_buf)
      pltpu.sync_copy(dst_hbm.at[pl.ds(ebase + j * CHUNK, CHUNK)], dst_buf)

      @pl.loop(0, VECS)
      def _(i):
        s = src_buf[pl.ds(i * 16, 16)]
        rows = i * 16 + iota
        for c in range(C):
          cvec = jnp.full((16,), c, jnp.int32)
          hv = plsc.load_gather(hw_loc, [s, cvec])
          plsc.store_scatter(vals, [rows, cvec], hv)

      pltpu.sync_copy(vals, acc.at[dst_buf], add=True)

    plsc.subcore_barrier()
    nbase = sid * SLICE
    pltpu.sync_copy(acc.at[pl.ds(nbase, SLICE)],
                    out_hbm.at[cid, pl.ds(nbase, SLICE)])

  return k


_count = _make_count()
_edge_apply = {c: _make_edge_apply(c) for c in (1, 2, 4)}


_VMEM_SPEC = pl.BlockSpec(memory_space=pltpu.VMEM)


def _node_prep_body(deg_ref, x_ref, w_ref, dinv_ref, hw_ref):
  deg = deg_ref[0, :, 0:1] + deg_ref[1, :, 0:1]
  dinv = lax.rsqrt(deg + 1.0)
  dinv_ref[...] = dinv
  h = jnp.dot(x_ref[...], w_ref[...], preferred_element_type=jnp.float32)
  hw_ref[...] = h * dinv


def _node_prep(deg, x_nm, Wp):
  return pl.pallas_call(
      _node_prep_body,
      out_shape=(
          jax.ShapeDtypeStruct((NPAD, 1), jnp.float32),
          jax.ShapeDtypeStruct((NPAD, LANES), jnp.float32),
      ),
      in_specs=[_VMEM_SPEC] * 3,
      out_specs=(_VMEM_SPEC,) * 2,
  )(deg, x_nm, Wp)


def _mix_body(acc_ref, hw_ref, dinv_ref, b_ref, w_ref, out_ref, *, C):
  acc = acc_ref[0, :, :C] + acc_ref[1, :, :C]
  x = (acc + hw_ref[:, :C]) * dinv_ref[...] + b_ref[:, :C]
  x = jnp.maximum(x, 0.0)
  h = jnp.dot(x, w_ref[:C, :], preferred_element_type=jnp.float32)
  out_ref[...] = h * dinv_ref[...]


def _mix(acc, hw, dinv, bp, Wp, C):
  return pl.pallas_call(
      functools.partial(_mix_body, C=C),
      out_shape=jax.ShapeDtypeStruct((NPAD, LANES), jnp.float32),
      in_specs=[_VMEM_SPEC] * 5,
      out_specs=_VMEM_SPEC,
  )(acc, hw, dinv, bp, Wp)


def _final_body(acc_ref, hw_ref, dinv_ref, b_ref, out_ref):
  acc = acc_ref[0, :, 0:1] + acc_ref[1, :, 0:1]
  out_ref[...] = (acc + hw_ref[:, 0:1]) * dinv_ref[...] + b_ref[...]


def _final(acc, hw, dinv, b3):
  return pl.pallas_call(
      _final_body,
      out_shape=jax.ShapeDtypeStruct((NPAD, 1), jnp.float32),
      in_specs=[_VMEM_SPEC] * 4,
      out_specs=_VMEM_SPEC,
  )(acc, hw, dinv, b3.reshape(1, 1))


_CB = 512


def _gemv1_body(x_ref, w_ref, b_ref, o_ref):
  o_ref[...] = jnp.tanh(
      jnp.dot(x_ref[...], w_ref[...], preferred_element_type=jnp.float32)
      + b_ref[...])


def _gemv1(x, W, b):
  kdim, mdim = W.shape
  ncol = pl.cdiv(mdim, _CB)
  return pl.pallas_call(
      _gemv1_body,
      grid=(ncol,),
      in_specs=[
          pl.BlockSpec((1, kdim), lambda i: (0, 0)),
          pl.BlockSpec((kdim, _CB), lambda i: (0, i)),
          pl.BlockSpec((1, _CB), lambda i: (0, i)),
      ],
      out_specs=pl.BlockSpec((1, _CB), lambda i: (0, i)),
      out_shape=jax.ShapeDtypeStruct((1, mdim), jnp.float32),
  )(x, W, b)


def _gemv2_body(x_ref, w_ref, b_ref, o_ref):
  o_ref[...] = jnp.tanh(
      jnp.dot(x_ref[...], w_ref[...], preferred_element_type=jnp.float32)
      + b_ref[...])


def _gemv2(x, W, b):
  return pl.pallas_call(
      _gemv2_body,
      out_shape=jax.ShapeDtypeStruct((1, W.shape[1]), jnp.float32),
  )(x, W, b)


def _pad_w(W):
  return jnp.zeros((LANES, LANES), jnp.float32).at[:W.shape[0],
                                                   :W.shape[1]].set(W)


def _pad_b(b):
  return jnp.zeros((1, LANES), jnp.float32).at[0, :b.shape[0]].set(b)


def kernel(data, edge, W1, b1, W2, b2, W3, b3, Wl1, bl1, Wl2, bl2):
  src = edge[0]
  dst = edge[1]
  zeros = jnp.zeros((NPAD, LANES), jnp.float32)
  x_nm = jnp.zeros((NPAD, LANES), jnp.float32).at[:N, :2].set(data)

  deg = _count(dst, zeros)
  dinv, hw1 = _node_prep(deg, x_nm, _pad_w(W1))

  acc1 = _edge_apply[4](src, dst, hw1[:, :4], zeros)
  hw2 = _mix(acc1, hw1, dinv, _pad_b(b1), _pad_w(W2), 4)

  acc2 = _edge_apply[2](src, dst, hw2[:, :2], zeros)
  hw3 = _mix(acc2, hw2, dinv, _pad_b(b2), _pad_w(W3), 2)

  acc3 = _edge_apply[1](src, dst, hw3[:, :1], zeros)
  v = _final(acc3, hw3, dinv, b3)

  x_vec = v.reshape(1, NPAD)[:, :N]
  hmid = _gemv1(x_vec, Wl1, bl1.reshape(1, -1))
  return _gemv2(hmid, Wl2, bl2.reshape(1, -1))


# speedup vs baseline: 40.6411x; 1.0134x over previous
import functools

import jax
import jax.numpy as jnp
from jax import lax
from jax.experimental import pallas as pl
from jax.experimental.pallas import tpu as pltpu
from jax.experimental.pallas import tpu_sc as plsc

N = 10000
E = 640000
NPAD = 10240
LANES = 16
NCORES = 2
NTILES = 16
NWORK = NCORES * NTILES
EPT = E // NWORK
CHUNK = 2000
NCHUNKS = EPT // CHUNK
VECS = CHUNK // 16
SLICE = NPAD // NTILES

_MESH = plsc.VectorSubcoreMesh(
    core_axis_name="c", subcore_axis_name="s", num_cores=NCORES)
_SC_PARAMS = pltpu.CompilerParams(
    needs_layout_passes=False, use_tc_tiling_on_sc=False)


def _make_count():
  @functools.partial(
      pl.kernel,
      out_type=jax.ShapeDtypeStruct((NCORES, NPAD, LANES), jnp.float32),
      mesh=_MESH,
      compiler_params=_SC_PARAMS,
      scratch_types=[
          pltpu.VMEM((CHUNK,), jnp.int32),
          pltpu.VMEM((CHUNK, LANES), jnp.float32),
          pltpu.VMEM_SHARED((NPAD, LANES), jnp.float32),
      ],
  )
  def k(dst_hbm, zeros_hbm, out_hbm, dst_buf, vals, acc):
    cid = lax.axis_index("c")
    sid = lax.axis_index("s")

    @pl.when(sid == 0)
    def _():
      pltpu.sync_copy(zeros_hbm, acc)

    pltpu.sync_copy(zeros_hbm.at[pl.ds(0, CHUNK)], vals)
    iota = lax.iota(jnp.int32, 16)
    zero16 = jnp.zeros((16,), jnp.int32)
    one16 = jnp.ones((16,), jnp.float32)

    @pl.loop(0, VECS)
    def _(i):
      plsc.store_scatter(vals, [i * 16 + iota, zero16], one16)

    plsc.subcore_barrier()
    ebase = (cid * NTILES + sid) * EPT

    @pl.loop(0, NCHUNKS)
    def _(j):
      pltpu.sync_copy(dst_hbm.at[pl.ds(ebase + j * CHUNK, CHUNK)], dst_buf)
      pltpu.sync_copy(vals, acc.at[dst_buf], add=True)

    plsc.subcore_barrier()
    nbase = sid * SLICE
    pltpu.sync_copy(acc.at[pl.ds(nbase, SLICE)],
                    out_hbm.at[cid, pl.ds(nbase, SLICE)])

  return k


def _make_edge_apply(C):
  @functools.partial(
      pl.kernel,
      out_type=jax.ShapeDtypeStruct((NCORES, NPAD, LANES), jnp.float32),
      mesh=_MESH,
      compiler_params=_SC_PARAMS,
      scratch_types=[
          pltpu.VMEM((NPAD, C), jnp.float32),
          pltpu.VMEM((CHUNK,), jnp.int32),
          pltpu.VMEM((CHUNK,), jnp.int32),
          pltpu.VMEM((CHUNK, LANES), jnp.float32),
          pltpu.VMEM_SHARED((NPAD, LANES), jnp.float32),
      ],
  )
  def k(src_hbm, dst_hbm, hw_hbm, zeros_hbm, out_hbm,
        hw_loc, src_buf, dst_buf, vals, acc):
    cid = lax.axis_index("c")
    sid = lax.axis_index("s")

    @pl.when(sid == 0)
    def _():
      pltpu.sync_copy(zeros_hbm, acc)

    pltpu.sync_copy(zeros_hbm.at[pl.ds(0, CHUNK)], vals)
    pltpu.sync_copy(hw_hbm, hw_loc)
    plsc.subcore_barrier()
    iota = lax.iota(jnp.int32, 16)
    ebase = (cid * NTILES + sid) * EPT

    @pl.loop(0, NCHUNKS)
    def _(j):
      pltpu.sync_copy(src_hbm.at[pl.ds(ebase + j * CHUNK, CHUNK)], src_buf)
      pltpu.sync_copy(dst_hbm.at[pl.ds(ebase + j * CHUNK, CHUNK)], dst_buf)

      @pl.loop(0, VECS)
      def _(i):
        s = src_buf[pl.ds(i * 16, 16)]
        rows = i * 16 + iota
        for c in range(C):
          cvec = jnp.full((16,), c, jnp.int32)
          hv = plsc.load_gather(hw_loc, [s, cvec])
          plsc.store_scatter(vals, [rows, cvec], hv)

      pltpu.sync_copy(vals, acc.at[dst_buf], add=True)

    plsc.subcore_barrier()
    nbase = sid * SLICE
    pltpu.sync_copy(acc.at[pl.ds(nbase, SLICE)],
                    out_hbm.at[cid, pl.ds(nbase, SLICE)])

  return k


_count = _make_count()
_edge_apply = {c: _make_edge_apply(c) for c in (1, 2, 4)}


_VMEM_SPEC = pl.BlockSpec(memory_space=pltpu.VMEM)


def _node_prep_body(deg_ref, x_ref, w_ref, dinv_ref, hw_ref):
  deg = deg_ref[0, :, 0:1] + deg_ref[1, :, 0:1]
  dinv = lax.rsqrt(deg + 1.0)
  dinv_ref[...] = dinv
  h = jnp.dot(x_ref[...], w_ref[...], preferred_element_type=jnp.float32)
  hw_ref[...] = h * dinv


def _node_prep(deg, x_nm, Wp):
  return pl.pallas_call(
      _node_prep_body,
      out_shape=(
          jax.ShapeDtypeStruct((NPAD, 1), jnp.float32),
          jax.ShapeDtypeStruct((NPAD, LANES), jnp.float32),
      ),
      in_specs=[_VMEM_SPEC] * 3,
      out_specs=(_VMEM_SPEC,) * 2,
  )(deg, x_nm, Wp)


def _mix_body(acc_ref, hw_ref, dinv_ref, b_ref, w_ref, out_ref, *, C):
  acc = acc_ref[0, :, :C] + acc_ref[1, :, :C]
  x = (acc + hw_ref[:, :C]) * dinv_ref[...] + b_ref[:, :C]
  x = jnp.maximum(x, 0.0)
  h = jnp.dot(x, w_ref[:C, :], preferred_element_type=jnp.float32)
  out_ref[...] = h * dinv_ref[...]


def _mix(acc, hw, dinv, bp, Wp, C):
  return pl.pallas_call(
      functools.partial(_mix_body, C=C),
      out_shape=jax.ShapeDtypeStruct((NPAD, LANES), jnp.float32),
      in_specs=[_VMEM_SPEC] * 5,
      out_specs=_VMEM_SPEC,
  )(acc, hw, dinv, bp, Wp)


def _final_body(acc_ref, hw_ref, dinv_ref, b_ref, out_ref):
  acc = acc_ref[0, :, 0:1] + acc_ref[1, :, 0:1]
  out_ref[...] = (acc + hw_ref[:, 0:1]) * dinv_ref[...] + b_ref[...]


def _final(acc, hw, dinv, b3):
  return pl.pallas_call(
      _final_body,
      out_shape=jax.ShapeDtypeStruct((NPAD, 1), jnp.float32),
      in_specs=[_VMEM_SPEC] * 4,
      out_specs=_VMEM_SPEC,
  )(acc, hw, dinv, b3.reshape(1, 1))


_CB = 640


def _gemv1_body(x_ref, w_ref, b_ref, o_ref):
  o_ref[...] = jnp.tanh(
      jnp.dot(x_ref[...], w_ref[...], preferred_element_type=jnp.float32)
      + b_ref[...])


def _gemv1(x, W, b):
  kdim, mdim = W.shape
  ncol = pl.cdiv(mdim, _CB)
  return pl.pallas_call(
      _gemv1_body,
      grid=(ncol,),
      in_specs=[
          pl.BlockSpec((1, kdim), lambda i: (0, 0)),
          pl.BlockSpec((kdim, _CB), lambda i: (0, i)),
          pl.BlockSpec((1, _CB), lambda i: (0, i)),
      ],
      out_specs=pl.BlockSpec((1, _CB), lambda i: (0, i)),
      out_shape=jax.ShapeDtypeStruct((1, mdim), jnp.float32),
  )(x, W, b)


def _gemv2_body(x_ref, w_ref, b_ref, o_ref):
  o_ref[...] = jnp.tanh(
      jnp.dot(x_ref[...], w_ref[...], preferred_element_type=jnp.float32)
      + b_ref[...])


def _gemv2(x, W, b):
  return pl.pallas_call(
      _gemv2_body,
      out_shape=jax.ShapeDtypeStruct((1, W.shape[1]), jnp.float32),
  )(x, W, b)


def _pad_w(W):
  return jnp.zeros((LANES, LANES), jnp.float32).at[:W.shape[0],
                                                   :W.shape[1]].set(W)


def _pad_b(b):
  return jnp.zeros((1, LANES), jnp.float32).at[0, :b.shape[0]].set(b)


def kernel(data, edge, W1, b1, W2, b2, W3, b3, Wl1, bl1, Wl2, bl2):
  src = edge[0]
  dst = edge[1]
  zeros = jnp.zeros((NPAD, LANES), jnp.float32)
  x_nm = jnp.zeros((NPAD, LANES), jnp.float32).at[:N, :2].set(data)

  deg = _count(dst, zeros)
  dinv, hw1 = _node_prep(deg, x_nm, _pad_w(W1))

  acc1 = _edge_apply[4](src, dst, hw1[:, :4], zeros)
  hw2 = _mix(acc1, hw1, dinv, _pad_b(b1), _pad_w(W2), 4)

  acc2 = _edge_apply[2](src, dst, hw2[:, :2], zeros)
  hw3 = _mix(acc2, hw2, dinv, _pad_b(b2), _pad_w(W3), 2)

  acc3 = _edge_apply[1](src, dst, hw3[:, :1], zeros)
  v = _final(acc3, hw3, dinv, b3)

  x_vec = v.reshape(1, NPAD)[:, :N]
  hmid = _gemv1(x_vec, Wl1, bl1.reshape(1, -1))
  return _gemv2(hmid, Wl2, bl2.reshape(1, -1))
